# baseline (speedup 1.0000x reference)
import functools

import jax
import jax.numpy as jnp
from jax import lax
from jax.experimental import pallas as pl
from jax.experimental.pallas import tpu as pltpu
from jax.experimental.pallas import tpu_sc as plsc

B, S, D = 4, 8192, 1024
R = 16
CH = R * D
L = 16


def _body(x_hbm, pe_hbm, out_hbm, xbuf, pebuf):
    nc = 2
    wid = lax.axis_index("s") * nc + lax.axis_index("c")
    rows_per_w = S // 32
    nchunk = rows_per_w // R

    def chunk_body(ci, _):
        s0 = wid * rows_per_w + ci * R
        pltpu.sync_copy(pe_hbm.at[pl.ds(s0 * D, CH)], pebuf)

        def batch_body(b, _):
            off = (b * S + s0) * D
            pltpu.sync_copy(x_hbm.at[pl.ds(off, CH)], xbuf)

            def add_body(i, _):
                base = i * (L * 8)
                for k in range(8):
                    sl = pl.ds(base + k * L, L)
                    xbuf[sl] = xbuf[sl] + pebuf[sl]
                return 0

            lax.fori_loop(0, CH // (L * 8), add_body, 0)
            pltpu.sync_copy(xbuf, out_hbm.at[pl.ds(off, CH)])
            return 0

        lax.fori_loop(0, B, batch_body, 0)
        return 0

    lax.fori_loop(0, nchunk, chunk_body, 0)


def kernel(x, pe_table):
    mesh = plsc.VectorSubcoreMesh(core_axis_name="c", subcore_axis_name="s")
    run = pl.kernel(
        _body,
        mesh=mesh,
        out_type=jax.ShapeDtypeStruct((B * S * D,), jnp.float32),
        scratch_types=[
            pltpu.VMEM((CH,), jnp.float32),
            pltpu.VMEM((CH,), jnp.float32),
        ],
    )
    out = run(x.reshape(B * S * D), pe_table.reshape(S * D))
    return out.reshape(B, S, D)
